# baseline (speedup 1.0000x reference)
import math

import jax
import jax.numpy as jnp
from jax import lax
from jax.experimental import pallas as pl
from jax.experimental.pallas import tpu as pltpu
from jax.experimental.pallas import tpu_sc as plsc

N = 10000
E = 320000
DIM = 128
FEATURE_LEN = 128
NSLOT = 4

NC = 2
NS = 16
NW = NC * NS

NP = 10240
CHUNK = 128
EW = 10240
NCH = EW // CHUNK
EP = NW * EW
RPT = NP // NS
KPT = RPT // CHUNK

_mesh = plsc.VectorSubcoreMesh(
    core_axis_name="c", subcore_axis_name="s", num_cores=NC, num_subcores=NS
)

_f32 = jnp.float32


def _sc_deg_body(src_hbm, dst_hbm, zeros_hbm, e0_hbm, e1_hbm,
                 degp_out, sidx, didx, vals0, vals1, acc_deg):
  cid = lax.axis_index("c")
  sid = lax.axis_index("s")
  wid = sid * NC + cid

  pltpu.sync_copy(zeros_hbm, vals0)
  for k in range(KPT):
    pltpu.sync_copy(vals0, acc_deg.at[pl.ds(sid * RPT + k * CHUNK, CHUNK)])
  pltpu.sync_copy(e0_hbm, vals0)
  pltpu.sync_copy(e1_hbm, vals1)
  plsc.subcore_barrier()

  @pl.loop(0, NCH)
  def _(j):
    base = wid * EW + j * CHUNK
    pltpu.sync_copy(dst_hbm.at[pl.ds(base, CHUNK)], didx)
    pltpu.sync_copy(vals0, acc_deg.at[didx], add=True)
    pltpu.sync_copy(src_hbm.at[pl.ds(base, CHUNK)], sidx)
    pltpu.sync_copy(vals1, acc_deg.at[sidx], add=True)

  plsc.subcore_barrier()
  for k in range(KPT):
    r0 = sid * RPT + k * CHUNK
    pltpu.sync_copy(acc_deg.at[pl.ds(r0, CHUNK)], vals0)
    pltpu.sync_copy(vals0, degp_out.at[cid, pl.ds(r0, CHUNK)])


def _sc_deg(src, dst, zeros, e0, e1):
  return pl.kernel(
      _sc_deg_body,
      out_type=jax.ShapeDtypeStruct((NC, NP, DIM), _f32),
      mesh=_mesh,
      scratch_types=[
          pltpu.VMEM((CHUNK,), jnp.int32),
          pltpu.VMEM((CHUNK,), jnp.int32),
          pltpu.VMEM((CHUNK, DIM), _f32),
          pltpu.VMEM((CHUNK, DIM), _f32),
          pltpu.VMEM_SHARED((NP, DIM), _f32),
      ],
  )(src, dst, zeros, e0, e1)


PCH = 32
NCH_SLOW = 32
NCH_FAST = 128
SLOW_CORE = 0


def _sc_edge_body(t_hbm, srcr_hbm, dstr_hbm, zeros_hbm, aggp_out,
                  idxb, rows0, rows1, semg0, semg1, acc):
  cid = lax.axis_index("c")
  sid = lax.axis_index("s")

  pltpu.sync_copy(zeros_hbm, rows0)
  for k in range(KPT):
    pltpu.sync_copy(rows0, acc.at[pl.ds(sid * RPT + k * CHUNK, CHUNK)])
  plsc.subcore_barrier()

  is_slow = cid == SLOW_CORE
  nph = jnp.where(is_slow, NCH_SLOW // PCH, NCH_FAST // PCH)
  cbase = jnp.where(is_slow, sid * NCH_SLOW,
                    NS * NCH_SLOW + sid * NCH_FAST)

  @pl.loop(0, nph)
  def _(p):
    cb = cbase + p * PCH
    pltpu.sync_copy(srcr_hbm.at[pl.ds(cb, PCH)], idxb.at[pl.ds(0, PCH)])
    pltpu.sync_copy(dstr_hbm.at[pl.ds(cb, PCH)], idxb.at[pl.ds(PCH, PCH)])
    pltpu.async_copy(t_hbm.at[idxb.at[0]], rows0, semg0)

    @pl.loop(0, PCH // 2)
    def _(j2):
      c0 = j2 * 2
      c1 = c0 + 1
      pltpu.async_copy(t_hbm.at[idxb.at[c1]], rows1, semg1)
      pltpu.make_async_copy(t_hbm.at[idxb.at[c0]], rows0, semg0).wait()
      pltpu.sync_copy(rows0, acc.at[idxb.at[PCH + c0]], add=True)

      @pl.when(c1 < PCH - 1)
      def _():
        pltpu.async_copy(t_hbm.at[idxb.at[c0 + 2]], rows0, semg0)

      pltpu.make_async_copy(t_hbm.at[idxb.at[c1]], rows1, semg1).wait()
      pltpu.sync_copy(rows1, acc.at[idxb.at[PCH + c1]], add=True)

  plsc.subcore_barrier()
  for k in range(KPT):
    r0 = sid * RPT + k * CHUNK
    pltpu.sync_copy(acc.at[pl.ds(r0, CHUNK)], rows0)
    pltpu.sync_copy(rows0, aggp_out.at[cid, pl.ds(r0, CHUNK)])


def _sc_edge(t, srcr, dstr, zeros):
  return pl.kernel(
      _sc_edge_body,
      out_type=jax.ShapeDtypeStruct((NC, NP, DIM), _f32),
      mesh=_mesh,
      scratch_types=[
          pltpu.VMEM((2 * PCH, DIM), jnp.int32),
          pltpu.VMEM((CHUNK, DIM), _f32),
          pltpu.VMEM((CHUNK, DIM), _f32),
          pltpu.SemaphoreType.DMA,
          pltpu.SemaphoreType.DMA,
          pltpu.VMEM_SHARED((NP, DIM), _f32),
      ],
  )(t, srcr, dstr, zeros)


def _tc_layer1_body(degp_ref, feat_ref, w1_ref, no_ref, ni_ref, t1_ref):
  d = degp_ref[...]
  deg_in = d[0, :, 0:1] + d[1, :, 0:1]
  deg_out = d[0, :, 1:2] + d[1, :, 1:2]
  no = jnp.where(deg_out > 0, lax.rsqrt(deg_out), 0.0)
  ni = jnp.where(deg_in > 0, lax.rsqrt(deg_in), 0.0)
  no_ref[...] = no
  ni_ref[...] = ni
  f = feat_ref[...]
  col = lax.broadcasted_iota(jnp.int32, (NP, FEATURE_LEN), 1)
  h0 = jnp.zeros((NP, FEATURE_LEN), _f32)
  for s in range(NSLOT):
    h0 = h0 + jnp.where(col == f[:, s:s + 1], 1.0, 0.0)
  t1_ref[...] = jnp.dot(h0 * no, w1_ref[...], preferred_element_type=_f32)


def _tc_layer1(degp, feat, w1):
  return pl.pallas_call(
      _tc_layer1_body,
      out_shape=[
          jax.ShapeDtypeStruct((NP, 1), _f32),
          jax.ShapeDtypeStruct((NP, 1), _f32),
          jax.ShapeDtypeStruct((NP, DIM), _f32),
      ],
  )(degp, feat, w1)


def _tc_layer2_body(aggp_ref, ni_ref, no_ref, b1_ref, w2_ref, t2_ref):
  a = aggp_ref[0] + aggp_ref[1]
  h1 = jnp.maximum(a * ni_ref[...] + b1_ref[...], 0.0)
  t2_ref[...] = jnp.dot(h1 * no_ref[...], w2_ref[...],
                        preferred_element_type=_f32)


def _tc_layer2(aggp, ni, no, b1, w2):
  return pl.pallas_call(
      _tc_layer2_body,
      out_shape=jax.ShapeDtypeStruct((NP, DIM), _f32),
  )(aggp, ni, no, b1, w2)


def _tc_final_body(aggp_ref, ni_ref, b2_ref, out_ref):
  a = aggp_ref[0] + aggp_ref[1]
  h2 = a * ni_ref[...] + b2_ref[...]
  valid = lax.broadcasted_iota(jnp.int32, (NP, 1), 0) < N
  h2 = jnp.where(valid, h2, 0.0)
  norms = jnp.sqrt(jnp.sum(h2 * h2, axis=1, keepdims=True))
  mean_norm = jnp.sum(norms) / N
  pooled = jnp.sum(h2, axis=0, keepdims=True)
  out_ref[...] = pooled * (math.sqrt(DIM) / mean_norm)


def _tc_final(aggp, ni, b2):
  return pl.pallas_call(
      _tc_final_body,
      out_shape=jax.ShapeDtypeStruct((1, DIM), _f32),
  )(aggp, ni, b2)


def kernel(feature, edge_index, W1, b1, W2, b2):
  feature = feature.astype(jnp.int32)
  edge_index = edge_index.astype(jnp.int32)
  src = edge_index[0]
  dst = edge_index[1]
  pad_e = jnp.full((EP - E,), N, jnp.int32)
  src = jnp.concatenate([src, pad_e])
  dst = jnp.concatenate([dst, pad_e])
  feat = jnp.pad(feature, ((0, NP - N), (0, 0)))

  lanes = jnp.arange(DIM)
  e0 = jnp.broadcast_to((lanes == 0).astype(_f32), (CHUNK, DIM))
  e1 = jnp.broadcast_to((lanes == 1).astype(_f32), (CHUNK, DIM))
  zeros128 = jnp.zeros((CHUNK, DIM), _f32)

  srcr = src.reshape(EP // CHUNK, CHUNK)
  dstr = dst.reshape(EP // CHUNK, CHUNK)

  degp = _sc_deg(src, dst, zeros128, e0, e1)
  no, ni, t1 = _tc_layer1(degp, feat, W1.astype(_f32))
  agg1 = _sc_edge(t1, srcr, dstr, zeros128)
  t2 = _tc_layer2(agg1, ni, no, b1.reshape(1, DIM).astype(_f32),
                  W2.astype(_f32))
  agg2 = _sc_edge(t2, srcr, dstr, zeros128)
  return _tc_final(agg2, ni, b2.reshape(1, DIM).astype(_f32))


# speedup vs baseline: 1.0007x; 1.0007x over previous
import math

import jax
import jax.numpy as jnp
from jax import lax
from jax.experimental import pallas as pl
from jax.experimental.pallas import tpu as pltpu
from jax.experimental.pallas import tpu_sc as plsc

N = 10000
E = 320000
DIM = 128
FEATURE_LEN = 128
NSLOT = 4

NC = 2
NS = 16
NW = NC * NS

NP = 10240
CHUNK = 128
EW = 10240
NCH = EW // CHUNK
EP = NW * EW
RPT = NP // NS
KPT = RPT // CHUNK

_mesh = plsc.VectorSubcoreMesh(
    core_axis_name="c", subcore_axis_name="s", num_cores=NC, num_subcores=NS
)

_f32 = jnp.float32


def _sc_deg_body(src_hbm, dst_hbm, zeros_hbm, e0_hbm, e1_hbm,
                 degp_out, sidx, didx, vals0, vals1, acc_deg):
  cid = lax.axis_index("c")
  sid = lax.axis_index("s")
  wid = sid * NC + cid

  pltpu.sync_copy(zeros_hbm, vals0)
  for k in range(KPT):
    pltpu.sync_copy(vals0, acc_deg.at[pl.ds(sid * RPT + k * CHUNK, CHUNK)])
  pltpu.sync_copy(e0_hbm, vals0)
  pltpu.sync_copy(e1_hbm, vals1)
  plsc.subcore_barrier()

  @pl.loop(0, NCH)
  def _(j):
    base = wid * EW + j * CHUNK
    pltpu.sync_copy(dst_hbm.at[pl.ds(base, CHUNK)], didx)
    pltpu.sync_copy(vals0, acc_deg.at[didx], add=True)
    pltpu.sync_copy(src_hbm.at[pl.ds(base, CHUNK)], sidx)
    pltpu.sync_copy(vals1, acc_deg.at[sidx], add=True)

  plsc.subcore_barrier()
  for k in range(KPT):
    r0 = sid * RPT + k * CHUNK
    pltpu.sync_copy(acc_deg.at[pl.ds(r0, CHUNK)], vals0)
    pltpu.sync_copy(vals0, degp_out.at[cid, pl.ds(r0, CHUNK)])


def _sc_deg(src, dst, zeros, e0, e1):
  return pl.kernel(
      _sc_deg_body,
      out_type=jax.ShapeDtypeStruct((NC, NP, DIM), _f32),
      mesh=_mesh,
      scratch_types=[
          pltpu.VMEM((CHUNK,), jnp.int32),
          pltpu.VMEM((CHUNK,), jnp.int32),
          pltpu.VMEM((CHUNK, DIM), _f32),
          pltpu.VMEM((CHUNK, DIM), _f32),
          pltpu.VMEM_SHARED((NP, DIM), _f32),
      ],
  )(src, dst, zeros, e0, e1)


PCH = 32
NCH_SLOW = 32
NCH_FAST = 128
SLOW_CORE = 1


def _sc_edge_body(t_hbm, srcr_hbm, dstr_hbm, zeros_hbm, aggp_out,
                  idxb, rows0, rows1, semg0, semg1, acc):
  cid = lax.axis_index("c")
  sid = lax.axis_index("s")

  pltpu.sync_copy(zeros_hbm, rows0)
  for k in range(KPT):
    pltpu.sync_copy(rows0, acc.at[pl.ds(sid * RPT + k * CHUNK, CHUNK)])
  plsc.subcore_barrier()

  is_slow = cid == SLOW_CORE
  nph = jnp.where(is_slow, NCH_SLOW // PCH, NCH_FAST // PCH)
  cbase = jnp.where(is_slow, sid * NCH_SLOW,
                    NS * NCH_SLOW + sid * NCH_FAST)

  @pl.loop(0, nph)
  def _(p):
    cb = cbase + p * PCH
    pltpu.sync_copy(srcr_hbm.at[pl.ds(cb, PCH)], idxb.at[pl.ds(0, PCH)])
    pltpu.sync_copy(dstr_hbm.at[pl.ds(cb, PCH)], idxb.at[pl.ds(PCH, PCH)])
    pltpu.async_copy(t_hbm.at[idxb.at[0]], rows0, semg0)

    @pl.loop(0, PCH // 2)
    def _(j2):
      c0 = j2 * 2
      c1 = c0 + 1
      pltpu.async_copy(t_hbm.at[idxb.at[c1]], rows1, semg1)
      pltpu.make_async_copy(t_hbm.at[idxb.at[c0]], rows0, semg0).wait()
      pltpu.sync_copy(rows0, acc.at[idxb.at[PCH + c0]], add=True)

      @pl.when(c1 < PCH - 1)
      def _():
        pltpu.async_copy(t_hbm.at[idxb.at[c0 + 2]], rows0, semg0)

      pltpu.make_async_copy(t_hbm.at[idxb.at[c1]], rows1, semg1).wait()
      pltpu.sync_copy(rows1, acc.at[idxb.at[PCH + c1]], add=True)

  plsc.subcore_barrier()
  for k in range(KPT):
    r0 = sid * RPT + k * CHUNK
    pltpu.sync_copy(acc.at[pl.ds(r0, CHUNK)], rows0)
    pltpu.sync_copy(rows0, aggp_out.at[cid, pl.ds(r0, CHUNK)])


def _sc_edge(t, srcr, dstr, zeros):
  return pl.kernel(
      _sc_edge_body,
      out_type=jax.ShapeDtypeStruct((NC, NP, DIM), _f32),
      mesh=_mesh,
      scratch_types=[
          pltpu.VMEM((2 * PCH, DIM), jnp.int32),
          pltpu.VMEM((CHUNK, DIM), _f32),
          pltpu.VMEM((CHUNK, DIM), _f32),
          pltpu.SemaphoreType.DMA,
          pltpu.SemaphoreType.DMA,
          pltpu.VMEM_SHARED((NP, DIM), _f32),
      ],
  )(t, srcr, dstr, zeros)


def _tc_layer1_body(degp_ref, feat_ref, w1_ref, no_ref, ni_ref, t1_ref):
  d = degp_ref[...]
  deg_in = d[0, :, 0:1] + d[1, :, 0:1]
  deg_out = d[0, :, 1:2] + d[1, :, 1:2]
  no = jnp.where(deg_out > 0, lax.rsqrt(deg_out), 0.0)
  ni = jnp.where(deg_in > 0, lax.rsqrt(deg_in), 0.0)
  no_ref[...] = no
  ni_ref[...] = ni
  f = feat_ref[...]
  col = lax.broadcasted_iota(jnp.int32, (NP, FEATURE_LEN), 1)
  h0 = jnp.zeros((NP, FEATURE_LEN), _f32)
  for s in range(NSLOT):
    h0 = h0 + jnp.where(col == f[:, s:s + 1], 1.0, 0.0)
  t1_ref[...] = jnp.dot(h0 * no, w1_ref[...], preferred_element_type=_f32)


def _tc_layer1(degp, feat, w1):
  return pl.pallas_call(
      _tc_layer1_body,
      out_shape=[
          jax.ShapeDtypeStruct((NP, 1), _f32),
          jax.ShapeDtypeStruct((NP, 1), _f32),
          jax.ShapeDtypeStruct((NP, DIM), _f32),
      ],
  )(degp, feat, w1)


def _tc_layer2_body(aggp_ref, ni_ref, no_ref, b1_ref, w2_ref, t2_ref):
  a = aggp_ref[0] + aggp_ref[1]
  h1 = jnp.maximum(a * ni_ref[...] + b1_ref[...], 0.0)
  t2_ref[...] = jnp.dot(h1 * no_ref[...], w2_ref[...],
                        preferred_element_type=_f32)


def _tc_layer2(aggp, ni, no, b1, w2):
  return pl.pallas_call(
      _tc_layer2_body,
      out_shape=jax.ShapeDtypeStruct((NP, DIM), _f32),
  )(aggp, ni, no, b1, w2)


def _tc_final_body(aggp_ref, ni_ref, b2_ref, out_ref):
  a = aggp_ref[0] + aggp_ref[1]
  h2 = a * ni_ref[...] + b2_ref[...]
  valid = lax.broadcasted_iota(jnp.int32, (NP, 1), 0) < N
  h2 = jnp.where(valid, h2, 0.0)
  norms = jnp.sqrt(jnp.sum(h2 * h2, axis=1, keepdims=True))
  mean_norm = jnp.sum(norms) / N
  pooled = jnp.sum(h2, axis=0, keepdims=True)
  out_ref[...] = pooled * (math.sqrt(DIM) / mean_norm)


def _tc_final(aggp, ni, b2):
  return pl.pallas_call(
      _tc_final_body,
      out_shape=jax.ShapeDtypeStruct((1, DIM), _f32),
  )(aggp, ni, b2)


def kernel(feature, edge_index, W1, b1, W2, b2):
  feature = feature.astype(jnp.int32)
  edge_index = edge_index.astype(jnp.int32)
  src = edge_index[0]
  dst = edge_index[1]
  pad_e = jnp.full((EP - E,), N, jnp.int32)
  src = jnp.concatenate([src, pad_e])
  dst = jnp.concatenate([dst, pad_e])
  feat = jnp.pad(feature, ((0, NP - N), (0, 0)))

  lanes = jnp.arange(DIM)
  e0 = jnp.broadcast_to((lanes == 0).astype(_f32), (CHUNK, DIM))
  e1 = jnp.broadcast_to((lanes == 1).astype(_f32), (CHUNK, DIM))
  zeros128 = jnp.zeros((CHUNK, DIM), _f32)

  srcr = src.reshape(EP // CHUNK, CHUNK)
  dstr = dst.reshape(EP // CHUNK, CHUNK)

  degp = _sc_deg(src, dst, zeros128, e0, e1)
  no, ni, t1 = _tc_layer1(degp, feat, W1.astype(_f32))
  agg1 = _sc_edge(t1, srcr, dstr, zeros128)
  t2 = _tc_layer2(agg1, ni, no, b1.reshape(1, DIM).astype(_f32),
                  W2.astype(_f32))
  agg2 = _sc_edge(t2, srcr, dstr, zeros128)
  return _tc_final(agg2, ni, b2.reshape(1, DIM).astype(_f32))


# speedup vs baseline: 1.0530x; 1.0522x over previous
import math

import jax
import jax.numpy as jnp
from jax import lax
from jax.experimental import pallas as pl
from jax.experimental.pallas import tpu as pltpu
from jax.experimental.pallas import tpu_sc as plsc

N = 10000
E = 320000
DIM = 128
FEATURE_LEN = 128
NSLOT = 4

NC = 2
NS = 16
NW = NC * NS

NP = 10240
CHUNK = 128
EW = 10240
NCH = EW // CHUNK
EP = NW * EW
RPT = NP // NS
KPT = RPT // CHUNK

_mesh = plsc.VectorSubcoreMesh(
    core_axis_name="c", subcore_axis_name="s", num_cores=NC, num_subcores=NS
)

_f32 = jnp.float32


def _sc_deg_body(src_hbm, dst_hbm, zeros_hbm, e0_hbm, e1_hbm,
                 degp_out, sidx, didx, vals0, vals1, acc_deg):
  cid = lax.axis_index("c")
  sid = lax.axis_index("s")
  wid = sid * NC + cid

  pltpu.sync_copy(zeros_hbm, vals0)
  for k in range(KPT):
    pltpu.sync_copy(vals0, acc_deg.at[pl.ds(sid * RPT + k * CHUNK, CHUNK)])
  pltpu.sync_copy(e0_hbm, vals0)
  pltpu.sync_copy(e1_hbm, vals1)
  plsc.subcore_barrier()

  @pl.loop(0, NCH)
  def _(j):
    base = wid * EW + j * CHUNK
    pltpu.sync_copy(dst_hbm.at[pl.ds(base, CHUNK)], didx)
    pltpu.sync_copy(vals0, acc_deg.at[didx], add=True)
    pltpu.sync_copy(src_hbm.at[pl.ds(base, CHUNK)], sidx)
    pltpu.sync_copy(vals1, acc_deg.at[sidx], add=True)

  plsc.subcore_barrier()
  for k in range(KPT):
    r0 = sid * RPT + k * CHUNK
    pltpu.sync_copy(acc_deg.at[pl.ds(r0, CHUNK)], vals0)
    pltpu.sync_copy(vals0, degp_out.at[cid, pl.ds(r0, CHUNK)])


def _sc_deg(src, dst, zeros, e0, e1):
  return pl.kernel(
      _sc_deg_body,
      out_type=jax.ShapeDtypeStruct((NC, NP, DIM), _f32),
      mesh=_mesh,
      scratch_types=[
          pltpu.VMEM((CHUNK,), jnp.int32),
          pltpu.VMEM((CHUNK,), jnp.int32),
          pltpu.VMEM((CHUNK, DIM), _f32),
          pltpu.VMEM((CHUNK, DIM), _f32),
          pltpu.VMEM_SHARED((NP, DIM), _f32),
      ],
  )(src, dst, zeros, e0, e1)


PCH = 40
NCH_SLOW = 80
NCH_FAST = 80
SLOW_CORE = 0


def _sc_edge_body(t_hbm, srcr_hbm, dstr_hbm, zeros_hbm, aggp_out,
                  idxb, rows0, rows1, semg0, semg1, acc):
  cid = lax.axis_index("c")
  sid = lax.axis_index("s")

  pltpu.sync_copy(zeros_hbm, rows0)
  for k in range(KPT):
    pltpu.sync_copy(rows0, acc.at[pl.ds(sid * RPT + k * CHUNK, CHUNK)])
  plsc.subcore_barrier()

  is_slow = cid == SLOW_CORE
  nph = jnp.where(is_slow, NCH_SLOW // PCH, NCH_FAST // PCH)
  cbase = jnp.where(is_slow, sid * NCH_SLOW,
                    NS * NCH_SLOW + sid * NCH_FAST)

  @pl.loop(0, nph)
  def _(p):
    cb = cbase + p * PCH
    pltpu.sync_copy(srcr_hbm.at[pl.ds(cb, PCH)], idxb.at[pl.ds(0, PCH)])
    pltpu.sync_copy(dstr_hbm.at[pl.ds(cb, PCH)], idxb.at[pl.ds(PCH, PCH)])
    pltpu.async_copy(t_hbm.at[idxb.at[0]], rows0, semg0)

    @pl.loop(0, PCH // 2)
    def _(j2):
      c0 = j2 * 2
      c1 = c0 + 1
      pltpu.async_copy(t_hbm.at[idxb.at[c1]], rows1, semg1)
      pltpu.make_async_copy(t_hbm.at[idxb.at[c0]], rows0, semg0).wait()
      pltpu.sync_copy(rows0, acc.at[idxb.at[PCH + c0]], add=True)

      @pl.when(c1 < PCH - 1)
      def _():
        pltpu.async_copy(t_hbm.at[idxb.at[c0 + 2]], rows0, semg0)

      pltpu.make_async_copy(t_hbm.at[idxb.at[c1]], rows1, semg1).wait()
      pltpu.sync_copy(rows1, acc.at[idxb.at[PCH + c1]], add=True)

  plsc.subcore_barrier()
  for k in range(KPT):
    r0 = sid * RPT + k * CHUNK
    pltpu.sync_copy(acc.at[pl.ds(r0, CHUNK)], rows0)
    pltpu.sync_copy(rows0, aggp_out.at[cid, pl.ds(r0, CHUNK)])


def _sc_edge(t, srcr, dstr, zeros):
  return pl.kernel(
      _sc_edge_body,
      out_type=jax.ShapeDtypeStruct((NC, NP, DIM), _f32),
      mesh=_mesh,
      scratch_types=[
          pltpu.VMEM((2 * PCH, DIM), jnp.int32),
          pltpu.VMEM((CHUNK, DIM), _f32),
          pltpu.VMEM((CHUNK, DIM), _f32),
          pltpu.SemaphoreType.DMA,
          pltpu.SemaphoreType.DMA,
          pltpu.VMEM_SHARED((NP, DIM), _f32),
      ],
  )(t, srcr, dstr, zeros)


def _tc_layer1_body(degp_ref, feat_ref, w1_ref, no_ref, ni_ref, t1_ref):
  d = degp_ref[...]
  deg_in = d[0, :, 0:1] + d[1, :, 0:1]
  deg_out = d[0, :, 1:2] + d[1, :, 1:2]
  no = jnp.where(deg_out > 0, lax.rsqrt(deg_out), 0.0)
  ni = jnp.where(deg_in > 0, lax.rsqrt(deg_in), 0.0)
  no_ref[...] = no
  ni_ref[...] = ni
  f = feat_ref[...]
  col = lax.broadcasted_iota(jnp.int32, (NP, FEATURE_LEN), 1)
  h0 = jnp.zeros((NP, FEATURE_LEN), _f32)
  for s in range(NSLOT):
    h0 = h0 + jnp.where(col == f[:, s:s + 1], 1.0, 0.0)
  t1_ref[...] = jnp.dot(h0 * no, w1_ref[...], preferred_element_type=_f32)


def _tc_layer1(degp, feat, w1):
  return pl.pallas_call(
      _tc_layer1_body,
      out_shape=[
          jax.ShapeDtypeStruct((NP, 1), _f32),
          jax.ShapeDtypeStruct((NP, 1), _f32),
          jax.ShapeDtypeStruct((NP, DIM), _f32),
      ],
  )(degp, feat, w1)


def _tc_layer2_body(aggp_ref, ni_ref, no_ref, b1_ref, w2_ref, t2_ref):
  a = aggp_ref[0] + aggp_ref[1]
  h1 = jnp.maximum(a * ni_ref[...] + b1_ref[...], 0.0)
  t2_ref[...] = jnp.dot(h1 * no_ref[...], w2_ref[...],
                        preferred_element_type=_f32)


def _tc_layer2(aggp, ni, no, b1, w2):
  return pl.pallas_call(
      _tc_layer2_body,
      out_shape=jax.ShapeDtypeStruct((NP, DIM), _f32),
  )(aggp, ni, no, b1, w2)


def _tc_final_body(aggp_ref, ni_ref, b2_ref, out_ref):
  a = aggp_ref[0] + aggp_ref[1]
  h2 = a * ni_ref[...] + b2_ref[...]
  valid = lax.broadcasted_iota(jnp.int32, (NP, 1), 0) < N
  h2 = jnp.where(valid, h2, 0.0)
  norms = jnp.sqrt(jnp.sum(h2 * h2, axis=1, keepdims=True))
  mean_norm = jnp.sum(norms) / N
  pooled = jnp.sum(h2, axis=0, keepdims=True)
  out_ref[...] = pooled * (math.sqrt(DIM) / mean_norm)


def _tc_final(aggp, ni, b2):
  return pl.pallas_call(
      _tc_final_body,
      out_shape=jax.ShapeDtypeStruct((1, DIM), _f32),
  )(aggp, ni, b2)


def kernel(feature, edge_index, W1, b1, W2, b2):
  feature = feature.astype(jnp.int32)
  edge_index = edge_index.astype(jnp.int32)
  src = edge_index[0]
  dst = edge_index[1]
  pad_e = jnp.full((EP - E,), N, jnp.int32)
  src = jnp.concatenate([src, pad_e])
  dst = jnp.concatenate([dst, pad_e])
  feat = jnp.pad(feature, ((0, NP - N), (0, 0)))

  lanes = jnp.arange(DIM)
  e0 = jnp.broadcast_to((lanes == 0).astype(_f32), (CHUNK, DIM))
  e1 = jnp.broadcast_to((lanes == 1).astype(_f32), (CHUNK, DIM))
  zeros128 = jnp.zeros((CHUNK, DIM), _f32)

  srcr = src.reshape(EP // CHUNK, CHUNK)
  dstr = dst.reshape(EP // CHUNK, CHUNK)

  degp = _sc_deg(src, dst, zeros128, e0, e1)
  no, ni, t1 = _tc_layer1(degp, feat, W1.astype(_f32))
  agg1 = _sc_edge(t1, srcr, dstr, zeros128)
  t2 = _tc_layer2(agg1, ni, no, b1.reshape(1, DIM).astype(_f32),
                  W2.astype(_f32))
  agg2 = _sc_edge(t2, srcr, dstr, zeros128)
  return _tc_final(agg2, ni, b2.reshape(1, DIM).astype(_f32))


# speedup vs baseline: 1.1207x; 1.0643x over previous
import math

import jax
import jax.numpy as jnp
from jax import lax
from jax.experimental import pallas as pl
from jax.experimental.pallas import tpu as pltpu
from jax.experimental.pallas import tpu_sc as plsc

N = 10000
E = 320000
DIM = 128
FEATURE_LEN = 128
NSLOT = 4

NC = 2
NS = 16
NW = NC * NS

NP = 10240
CHUNK = 128
EW = 10240
NCH = EW // CHUNK
EP = NW * EW
RPT = NP // NS
KPT = RPT // CHUNK

_mesh = plsc.VectorSubcoreMesh(
    core_axis_name="c", subcore_axis_name="s", num_cores=NC, num_subcores=NS
)

_f32 = jnp.float32


def _sc_deg_body(src_hbm, dst_hbm, zeros_hbm, e0_hbm, e1_hbm,
                 degp_out, sidx, didx, vals0, vals1, acc_deg):
  cid = lax.axis_index("c")
  sid = lax.axis_index("s")
  wid = sid * NC + cid

  pltpu.sync_copy(zeros_hbm, vals0)
  for k in range(KPT):
    pltpu.sync_copy(vals0, acc_deg.at[pl.ds(sid * RPT + k * CHUNK, CHUNK)])
  pltpu.sync_copy(e0_hbm, vals0)
  pltpu.sync_copy(e1_hbm, vals1)
  plsc.subcore_barrier()

  @pl.loop(0, NCH)
  def _(j):
    base = wid * EW + j * CHUNK
    pltpu.sync_copy(dst_hbm.at[pl.ds(base, CHUNK)], didx)
    pltpu.sync_copy(vals0, acc_deg.at[didx], add=True)
    pltpu.sync_copy(src_hbm.at[pl.ds(base, CHUNK)], sidx)
    pltpu.sync_copy(vals1, acc_deg.at[sidx], add=True)

  plsc.subcore_barrier()
  for k in range(KPT):
    r0 = sid * RPT + k * CHUNK
    pltpu.sync_copy(acc_deg.at[pl.ds(r0, CHUNK)], vals0)
    pltpu.sync_copy(vals0, degp_out.at[cid, pl.ds(r0, CHUNK)])


def _sc_deg(src, dst, zeros, e0, e1):
  return pl.kernel(
      _sc_deg_body,
      out_type=jax.ShapeDtypeStruct((NC, NP, DIM), _f32),
      mesh=_mesh,
      scratch_types=[
          pltpu.VMEM((CHUNK,), jnp.int32),
          pltpu.VMEM((CHUNK,), jnp.int32),
          pltpu.VMEM((CHUNK, DIM), _f32),
          pltpu.VMEM((CHUNK, DIM), _f32),
          pltpu.VMEM_SHARED((NP, DIM), _f32),
      ],
  )(src, dst, zeros, e0, e1)


MCH = NCH // 2


def _sc_edge_body(t_hbm, srcr_hbm, dstr_hbm, zeros_hbm, aggp_out,
                  idxb, rows0, rows1, semg0, semg1, acc):
  cid = lax.axis_index("c")
  sid = lax.axis_index("s")
  wid = sid * NC + cid

  pltpu.sync_copy(zeros_hbm, rows0)
  for k in range(KPT):
    pltpu.sync_copy(rows0, acc.at[pl.ds(sid * RPT + k * CHUNK, CHUNK)])
  plsc.subcore_barrier()

  for mp in range(2):
    pltpu.sync_copy(srcr_hbm.at[wid, pl.ds(mp * MCH, MCH)],
                    idxb.at[pl.ds(0, MCH)])
    pltpu.sync_copy(dstr_hbm.at[wid, pl.ds(mp * MCH, MCH)],
                    idxb.at[pl.ds(MCH, MCH)])
    pltpu.async_copy(t_hbm.at[idxb.at[0]], rows0, semg0)

    @pl.loop(0, MCH // 2)
    def _(j2):
      c0 = j2 * 2
      c1 = c0 + 1
      pltpu.async_copy(t_hbm.at[idxb.at[c1]], rows1, semg1)
      pltpu.make_async_copy(t_hbm.at[idxb.at[c0]], rows0, semg0).wait()
      pltpu.sync_copy(rows0, acc.at[idxb.at[MCH + c0]], add=True)

      @pl.when(c1 < MCH - 1)
      def _():
        pltpu.async_copy(t_hbm.at[idxb.at[c0 + 2]], rows0, semg0)

      pltpu.make_async_copy(t_hbm.at[idxb.at[c1]], rows1, semg1).wait()
      pltpu.sync_copy(rows1, acc.at[idxb.at[MCH + c1]], add=True)

  plsc.subcore_barrier()
  for k in range(KPT):
    r0 = sid * RPT + k * CHUNK
    pltpu.sync_copy(acc.at[pl.ds(r0, CHUNK)], rows0)
    pltpu.sync_copy(rows0, aggp_out.at[cid, pl.ds(r0, CHUNK)])


def _sc_edge(t, srcr, dstr, zeros):
  return pl.kernel(
      _sc_edge_body,
      out_type=jax.ShapeDtypeStruct((NC, NP, DIM), _f32),
      mesh=_mesh,
      scratch_types=[
          pltpu.VMEM((2 * MCH, DIM), jnp.int32),
          pltpu.VMEM((CHUNK, DIM), _f32),
          pltpu.VMEM((CHUNK, DIM), _f32),
          pltpu.SemaphoreType.DMA,
          pltpu.SemaphoreType.DMA,
          pltpu.VMEM_SHARED((NP, DIM), _f32),
      ],
  )(t, srcr, dstr, zeros)


def _tc_layer1_body(degp_ref, feat_ref, w1_ref, no_ref, ni_ref, t1_ref):
  d = degp_ref[...]
  deg_in = d[0, :, 0:1] + d[1, :, 0:1]
  deg_out = d[0, :, 1:2] + d[1, :, 1:2]
  no = jnp.where(deg_out > 0, lax.rsqrt(deg_out), 0.0)
  ni = jnp.where(deg_in > 0, lax.rsqrt(deg_in), 0.0)
  no_ref[...] = no
  ni_ref[...] = ni
  f = feat_ref[...]
  col = lax.broadcasted_iota(jnp.int32, (NP, FEATURE_LEN), 1)
  h0 = jnp.zeros((NP, FEATURE_LEN), _f32)
  for s in range(NSLOT):
    h0 = h0 + jnp.where(col == f[:, s:s + 1], 1.0, 0.0)
  t1_ref[...] = jnp.dot(h0 * no, w1_ref[...], preferred_element_type=_f32)


def _tc_layer1(degp, feat, w1):
  return pl.pallas_call(
      _tc_layer1_body,
      out_shape=[
          jax.ShapeDtypeStruct((NP, 1), _f32),
          jax.ShapeDtypeStruct((NP, 1), _f32),
          jax.ShapeDtypeStruct((NP, DIM), _f32),
      ],
  )(degp, feat, w1)


def _tc_layer2_body(aggp_ref, ni_ref, no_ref, b1_ref, w2_ref, t2_ref):
  a = aggp_ref[0] + aggp_ref[1]
  h1 = jnp.maximum(a * ni_ref[...] + b1_ref[...], 0.0)
  t2_ref[...] = jnp.dot(h1 * no_ref[...], w2_ref[...],
                        preferred_element_type=_f32)


def _tc_layer2(aggp, ni, no, b1, w2):
  return pl.pallas_call(
      _tc_layer2_body,
      out_shape=jax.ShapeDtypeStruct((NP, DIM), _f32),
  )(aggp, ni, no, b1, w2)


def _tc_final_body(aggp_ref, ni_ref, b2_ref, out_ref):
  a = aggp_ref[0] + aggp_ref[1]
  h2 = a * ni_ref[...] + b2_ref[...]
  valid = lax.broadcasted_iota(jnp.int32, (NP, 1), 0) < N
  h2 = jnp.where(valid, h2, 0.0)
  norms = jnp.sqrt(jnp.sum(h2 * h2, axis=1, keepdims=True))
  mean_norm = jnp.sum(norms) / N
  pooled = jnp.sum(h2, axis=0, keepdims=True)
  out_ref[...] = pooled * (math.sqrt(DIM) / mean_norm)


def _tc_final(aggp, ni, b2):
  return pl.pallas_call(
      _tc_final_body,
      out_shape=jax.ShapeDtypeStruct((1, DIM), _f32),
  )(aggp, ni, b2)


def kernel(feature, edge_index, W1, b1, W2, b2):
  feature = feature.astype(jnp.int32)
  edge_index = edge_index.astype(jnp.int32)
  src = edge_index[0]
  dst = edge_index[1]
  pad_e = jnp.full((EP - E,), N, jnp.int32)
  src = jnp.concatenate([src, pad_e])
  dst = jnp.concatenate([dst, pad_e])
  feat = jnp.pad(feature, ((0, NP - N), (0, 0)))

  lanes = jnp.arange(DIM)
  e0 = jnp.broadcast_to((lanes == 0).astype(_f32), (CHUNK, DIM))
  e1 = jnp.broadcast_to((lanes == 1).astype(_f32), (CHUNK, DIM))
  zeros128 = jnp.zeros((CHUNK, DIM), _f32)

  srcr = src.reshape(NW, NCH, CHUNK)
  dstr = dst.reshape(NW, NCH, CHUNK)

  degp = _sc_deg(src, dst, zeros128, e0, e1)
  no, ni, t1 = _tc_layer1(degp, feat, W1.astype(_f32))
  agg1 = _sc_edge(t1, srcr, dstr, zeros128)
  t2 = _tc_layer2(agg1, ni, no, b1.reshape(1, DIM).astype(_f32),
                  W2.astype(_f32))
  agg2 = _sc_edge(t2, srcr, dstr, zeros128)
  return _tc_final(agg2, ni, b2.reshape(1, DIM).astype(_f32))
